# baseline (speedup 1.0000x reference)
import functools

import jax
import jax.numpy as jnp
from jax import lax
from jax.experimental import pallas as pl
from jax.experimental.pallas import tpu as pltpu
from jax.experimental.pallas import tpu_sc as plsc


def kernel(indices, table):
    B, T = indices.shape
    V, D = table.shape
    info = plsc.get_sparse_core_info()
    NC, NS = info.num_cores, info.num_subcores
    NW = NC * NS
    rows_w = B // NW
    assert rows_w * NW == B

    mesh = plsc.VectorSubcoreMesh(core_axis_name="c", subcore_axis_name="s")

    @functools.partial(
        pl.kernel,
        mesh=mesh,
        out_type=jax.ShapeDtypeStruct((B, T, D), jnp.float32),
        scratch_types=[
            pltpu.VMEM((rows_w, T), jnp.int32),
            pltpu.VMEM((2, T, D), jnp.float32),
            pltpu.SemaphoreType.DMA,
        ],
        compiler_params=pltpu.CompilerParams(use_tc_tiling_on_sc=False),
    )
    def run(idx_hbm, tab_hbm, out_hbm, idx_v, buf, sem):
        wid = lax.axis_index("s") * NC + lax.axis_index("c")
        base = wid * rows_w
        pltpu.sync_copy(idx_hbm.at[pl.ds(base, rows_w)], idx_v)

        pltpu.async_copy(tab_hbm.at[idx_v.at[0]], buf.at[0], sem)

        def body(i, carry):
            p = lax.rem(i, 2)
            pltpu.make_async_copy(tab_hbm.at[idx_v.at[i]], buf.at[p], sem).wait()

            @pl.when(i + 1 < rows_w)
            def _():
                pltpu.async_copy(tab_hbm.at[idx_v.at[i + 1]], buf.at[1 - p], sem)

            pltpu.sync_copy(buf.at[p], out_hbm.at[base + i])
            return carry

        lax.fori_loop(0, rows_w, body, 0)

    return run(indices.astype(jnp.int32), table)
